# baseline (speedup 1.0000x reference)
import functools

import jax
import jax.numpy as jnp
from jax import lax
from jax.experimental import pallas as pl
from jax.experimental.pallas import tpu as pltpu
from jax.experimental.pallas import tpu_sc as plsc

D_MODEL = 64
ROW = 128
G = 4
C = G * ROW


def _emb_sc(idx2d, table, n_rows_per_worker):
    n_rows_total = idx2d.shape[0]
    b_total = n_rows_total * ROW
    info = plsc.get_sparse_core_info()
    nc, ns = info.num_cores, info.num_subcores
    ngroups = n_rows_per_worker // G
    assert ngroups % 2 == 0 and ngroups >= 4

    mesh = plsc.VectorSubcoreMesh(core_axis_name="c", subcore_axis_name="s")

    @functools.partial(
        pl.kernel,
        mesh=mesh,
        compiler_params=pltpu.CompilerParams(use_tc_tiling_on_sc=False),
        out_type=jax.ShapeDtypeStruct((b_total, D_MODEL), jnp.float32),
        scratch_types=[
            pltpu.VMEM((2 * G, ROW), jnp.int32),
            pltpu.VMEM((C, D_MODEL), jnp.float32),
            pltpu.VMEM((C, D_MODEL), jnp.float32),
            pltpu.SemaphoreType.DMA,
            pltpu.SemaphoreType.DMA,
            pltpu.SemaphoreType.DMA,
            pltpu.SemaphoreType.DMA,
        ],
    )
    def k(table_hbm, idx_hbm, out_hbm, idx_v, rows0, rows1, g0, g1, w0, w1):
        rows = (rows0, rows1)
        gsem = (g0, g1)
        wsem = (w0, w1)
        wid = lax.axis_index("s") * nc + lax.axis_index("c")
        row_base = wid * n_rows_per_worker

        def load_idx(i):
            pltpu.sync_copy(idx_hbm.at[pl.ds(row_base + (2 * G) * i, 2 * G)],
                            idx_v)

        def fire_gathers(b):
            for j in range(G):
                pltpu.async_copy(table_hbm.at[idx_v.at[b * G + j]],
                                 rows[b].at[pl.ds(j * ROW, ROW)], gsem[b])

        def wait_gathers(b):
            for j in range(G):
                pltpu.make_async_copy(table_hbm.at[idx_v.at[b * G + j]],
                                      rows[b].at[pl.ds(j * ROW, ROW)],
                                      gsem[b]).wait()

        def fire_write(b, g):
            pltpu.async_copy(rows[b],
                             out_hbm.at[pl.ds((row_base + g * G) * ROW, C)],
                             wsem[b])

        def wait_write(b):
            pltpu.make_async_copy(rows[b],
                                  out_hbm.at[pl.ds(row_base * ROW, C)],
                                  wsem[b]).wait()

        load_idx(0)
        fire_gathers(0)
        fire_gathers(1)
        wait_gathers(0)
        fire_write(0, 0)
        wait_gathers(1)
        fire_write(1, 1)

        def body(i, _):
            load_idx(i)
            wait_write(0)
            fire_gathers(0)
            wait_write(1)
            fire_gathers(1)
            wait_gathers(0)
            fire_write(0, 2 * i)
            wait_gathers(1)
            fire_write(1, 2 * i + 1)
            return ()

        lax.fori_loop(1, ngroups // 2, body, ())
        wait_write(0)
        wait_write(1)

    return k(table, idx2d)


def kernel(x, emb_weight):
    b, t = x.shape
    b_total = b * t
    n_rows = b_total // ROW
    info = plsc.get_sparse_core_info()
    nw = info.num_cores * info.num_subcores
    n_rows_per_worker = n_rows // nw
    assert n_rows_per_worker * nw == n_rows and n_rows_per_worker % G == 0

    idx2d = x.reshape(n_rows, ROW).astype(jnp.int32)
    out = _emb_sc(idx2d, emb_weight, n_rows_per_worker)
    return out.reshape(b, t, D_MODEL)


# speedup vs baseline: 1.5990x; 1.5990x over previous
import functools

import jax
import jax.numpy as jnp
from jax import lax
from jax.experimental import pallas as pl
from jax.experimental.pallas import tpu as pltpu
from jax.experimental.pallas import tpu_sc as plsc

D_MODEL = 64
ROW = 128
G = 4
C = G * ROW


def _emb_sc(idx2d, table, n_rows_per_worker):
    n_rows_total = idx2d.shape[0]
    b_total = n_rows_total * ROW
    info = plsc.get_sparse_core_info()
    nc, ns = info.num_cores, info.num_subcores
    ngroups = n_rows_per_worker // G
    assert ngroups % 2 == 0 and ngroups >= 4

    mesh = plsc.VectorSubcoreMesh(core_axis_name="c", subcore_axis_name="s")

    @functools.partial(
        pl.kernel,
        mesh=mesh,
        compiler_params=pltpu.CompilerParams(use_tc_tiling_on_sc=False),
        out_type=jax.ShapeDtypeStruct((b_total, D_MODEL), jnp.float32),
        scratch_types=[
            pltpu.VMEM((2 * G, ROW), jnp.int32),
            pltpu.VMEM((C, D_MODEL), jnp.float32),
            pltpu.VMEM((C, D_MODEL), jnp.float32),
            pltpu.VMEM_SHARED((table.shape[0], D_MODEL), jnp.float32),
            pltpu.SemaphoreType.DMA,
            pltpu.SemaphoreType.DMA,
            pltpu.SemaphoreType.DMA,
            pltpu.SemaphoreType.DMA,
        ],
    )
    def k(table_hbm, idx_hbm, out_hbm, idx_v, rows0, rows1, table_sp,
          g0, g1, w0, w1):
        rows = (rows0, rows1)
        gsem = (g0, g1)
        wsem = (w0, w1)
        wid = lax.axis_index("s") * nc + lax.axis_index("c")
        row_base = wid * n_rows_per_worker

        def load_idx(i):
            pltpu.sync_copy(idx_hbm.at[pl.ds(row_base + (2 * G) * i, 2 * G)],
                            idx_v)

        def fire_gathers(b):
            for j in range(G):
                pltpu.async_copy(table_sp.at[idx_v.at[b * G + j]],
                                 rows[b].at[pl.ds(j * ROW, ROW)], gsem[b])

        def wait_gathers(b):
            for j in range(G):
                pltpu.make_async_copy(table_sp.at[idx_v.at[b * G + j]],
                                      rows[b].at[pl.ds(j * ROW, ROW)],
                                      gsem[b]).wait()

        def fire_write(b, g):
            pltpu.async_copy(rows[b],
                             out_hbm.at[pl.ds((row_base + g * G) * ROW, C)],
                             wsem[b])

        def wait_write(b):
            pltpu.make_async_copy(rows[b],
                                  out_hbm.at[pl.ds(row_base * ROW, C)],
                                  wsem[b]).wait()

        @pl.when(lax.axis_index("s") == 0)
        def _():
            pltpu.sync_copy(table_hbm, table_sp)

        plsc.subcore_barrier()

        load_idx(0)
        fire_gathers(0)
        fire_gathers(1)
        wait_gathers(0)
        fire_write(0, 0)
        wait_gathers(1)
        fire_write(1, 1)

        def body(i, _):
            load_idx(i)
            wait_write(0)
            fire_gathers(0)
            wait_write(1)
            fire_gathers(1)
            wait_gathers(0)
            fire_write(0, 2 * i)
            wait_gathers(1)
            fire_write(1, 2 * i + 1)
            return ()

        lax.fori_loop(1, ngroups // 2, body, ())
        wait_write(0)
        wait_write(1)

    return k(table, idx2d)


def kernel(x, emb_weight):
    b, t = x.shape
    b_total = b * t
    n_rows = b_total // ROW
    info = plsc.get_sparse_core_info()
    nw = info.num_cores * info.num_subcores
    n_rows_per_worker = n_rows // nw
    assert n_rows_per_worker * nw == n_rows and n_rows_per_worker % G == 0

    idx2d = x.reshape(n_rows, ROW).astype(jnp.int32)
    out = _emb_sc(idx2d, emb_weight, n_rows_per_worker)
    return out.reshape(b, t, D_MODEL)


# speedup vs baseline: 1.6012x; 1.0013x over previous
import functools

import jax
import jax.numpy as jnp
from jax import lax
from jax.experimental import pallas as pl
from jax.experimental.pallas import tpu as pltpu
from jax.experimental.pallas import tpu_sc as plsc

D_MODEL = 64
ROW = 512
G = 1
C = G * ROW


def _emb_sc(idx2d, table, n_rows_per_worker):
    n_rows_total = idx2d.shape[0]
    b_total = n_rows_total * ROW
    info = plsc.get_sparse_core_info()
    nc, ns = info.num_cores, info.num_subcores
    ngroups = n_rows_per_worker // G
    assert ngroups % 2 == 0 and ngroups >= 4

    mesh = plsc.VectorSubcoreMesh(core_axis_name="c", subcore_axis_name="s")

    @functools.partial(
        pl.kernel,
        mesh=mesh,
        compiler_params=pltpu.CompilerParams(use_tc_tiling_on_sc=False),
        out_type=jax.ShapeDtypeStruct((b_total, D_MODEL), jnp.float32),
        scratch_types=[
            pltpu.VMEM((2 * G, ROW), jnp.int32),
            pltpu.VMEM((C, D_MODEL), jnp.float32),
            pltpu.VMEM((C, D_MODEL), jnp.float32),
            pltpu.VMEM_SHARED((table.shape[0], D_MODEL), jnp.float32),
            pltpu.SemaphoreType.DMA,
            pltpu.SemaphoreType.DMA,
            pltpu.SemaphoreType.DMA,
            pltpu.SemaphoreType.DMA,
        ],
    )
    def k(table_hbm, idx_hbm, out_hbm, idx_v, rows0, rows1, table_sp,
          g0, g1, w0, w1):
        rows = (rows0, rows1)
        gsem = (g0, g1)
        wsem = (w0, w1)
        wid = lax.axis_index("s") * nc + lax.axis_index("c")
        row_base = wid * n_rows_per_worker

        def load_idx(i):
            pltpu.sync_copy(idx_hbm.at[pl.ds(row_base + (2 * G) * i, 2 * G)],
                            idx_v)

        def fire_gathers(b):
            for j in range(G):
                pltpu.async_copy(table_sp.at[idx_v.at[b * G + j]],
                                 rows[b].at[pl.ds(j * ROW, ROW)], gsem[b])

        def wait_gathers(b):
            for j in range(G):
                pltpu.make_async_copy(table_sp.at[idx_v.at[b * G + j]],
                                      rows[b].at[pl.ds(j * ROW, ROW)],
                                      gsem[b]).wait()

        def fire_write(b, g):
            pltpu.async_copy(rows[b],
                             out_hbm.at[pl.ds((row_base + g * G) * ROW, C)],
                             wsem[b])

        def wait_write(b):
            pltpu.make_async_copy(rows[b],
                                  out_hbm.at[pl.ds(row_base * ROW, C)],
                                  wsem[b]).wait()

        @pl.when(lax.axis_index("s") == 0)
        def _():
            pltpu.sync_copy(table_hbm, table_sp)

        plsc.subcore_barrier()

        load_idx(0)
        fire_gathers(0)
        fire_gathers(1)
        wait_gathers(0)
        fire_write(0, 0)
        wait_gathers(1)
        fire_write(1, 1)

        def body(i, _):
            load_idx(i)
            wait_write(0)
            fire_gathers(0)
            wait_write(1)
            fire_gathers(1)
            wait_gathers(0)
            fire_write(0, 2 * i)
            wait_gathers(1)
            fire_write(1, 2 * i + 1)
            return ()

        lax.fori_loop(1, ngroups // 2, body, ())
        wait_write(0)
        wait_write(1)

    return k(table, idx2d)


def kernel(x, emb_weight):
    b, t = x.shape
    b_total = b * t
    n_rows = b_total // ROW
    info = plsc.get_sparse_core_info()
    nw = info.num_cores * info.num_subcores
    n_rows_per_worker = n_rows // nw
    assert n_rows_per_worker * nw == n_rows and n_rows_per_worker % G == 0

    idx2d = x.reshape(n_rows, ROW).astype(jnp.int32)
    out = _emb_sc(idx2d, emb_weight, n_rows_per_worker)
    return out.reshape(b, t, D_MODEL)


# speedup vs baseline: 2.0895x; 1.3050x over previous
import functools

import jax
import jax.numpy as jnp
from jax import lax
from jax.experimental import pallas as pl
from jax.experimental.pallas import tpu as pltpu
from jax.experimental.pallas import tpu_sc as plsc

D_MODEL = 64
VOCAB = 288
CB = 512
LG = CB // 16


def _emb_t(x_t, table_flat):
    T, B = x_t.shape
    info = plsc.get_sparse_core_info()
    nc, ns = info.num_cores, info.num_subcores
    nw = nc * ns
    n_units = T * (B // CB)
    units_per_worker = n_units // nw
    assert units_per_worker * nw == n_units and units_per_worker % 2 == 0
    nchunks = B // CB

    mesh = plsc.VectorSubcoreMesh(core_axis_name="c", subcore_axis_name="s")

    @functools.partial(
        pl.kernel,
        mesh=mesh,
        compiler_params=pltpu.CompilerParams(needs_layout_passes=False),
        out_type=jax.ShapeDtypeStruct((T, D_MODEL, B), jnp.float32),
        scratch_types=[
            pltpu.VMEM((D_MODEL * VOCAB,), jnp.float32),
            pltpu.VMEM((CB,), jnp.int32),
            pltpu.VMEM((CB,), jnp.int32),
            pltpu.VMEM((D_MODEL, CB), jnp.float32),
            pltpu.VMEM((D_MODEL, CB), jnp.float32),
            pltpu.SemaphoreType.DMA,
            pltpu.SemaphoreType.DMA,
        ],
    )
    def k(xt_hbm, tab_hbm, out_hbm, tab_v, idx0, idx1, buf0, buf1, w0, w1):
        idx = (idx0, idx1)
        buf = (buf0, buf1)
        wsem = (w0, w1)
        wid = lax.axis_index("s") * nc + lax.axis_index("c")
        u_base = wid * units_per_worker

        pltpu.sync_copy(tab_hbm, tab_v)

        def load_idx(p, u):
            t = u // nchunks
            b0 = (u % nchunks) * CB
            pltpu.sync_copy(xt_hbm.at[t, pl.ds(b0, CB)], idx[p])

        def compute(p):
            def body(g, _):
                iv = idx[p][pl.ds(g * 16, 16)]
                for d in range(D_MODEL):
                    v = plsc.load_gather(tab_v, [iv + d * VOCAB])
                    buf[p][d, pl.ds(g * 16, 16)] = v
                return ()

            lax.fori_loop(0, LG, body, ())

        def fire_write(p, u):
            t = u // nchunks
            b0 = (u % nchunks) * CB
            pltpu.async_copy(buf[p], out_hbm.at[t, :, pl.ds(b0, CB)],
                             wsem[p])

        def wait_write(p):
            pltpu.make_async_copy(buf[p], out_hbm.at[0, :, pl.ds(0, CB)],
                                  wsem[p]).wait()

        load_idx(0, u_base)
        compute(0)
        fire_write(0, u_base)
        load_idx(1, u_base + 1)
        compute(1)
        fire_write(1, u_base + 1)

        def body(i, _):
            for p in range(2):
                u = u_base + 2 * i + p
                load_idx(p, u)
                wait_write(p)
                compute(p)
                fire_write(p, u)
            return ()

        lax.fori_loop(1, units_per_worker // 2, body, ())
        wait_write(0)
        wait_write(1)

    return k(x_t, table_flat)


def kernel(x, emb_weight):
    b, t = x.shape
    x_t = x.T
    table_flat = emb_weight.T.reshape(D_MODEL * VOCAB)
    out_t = _emb_t(x_t, table_flat)
    return jnp.transpose(out_t, (2, 0, 1))


# speedup vs baseline: 9.1147x; 4.3622x over previous
import functools

import jax
import jax.numpy as jnp
from jax import lax
from jax.experimental import pallas as pl
from jax.experimental.pallas import tpu as pltpu
from jax.experimental.pallas import tpu_sc as plsc

D_MODEL = 64
VOCAB = 288
CB = 512
LG = CB // 16


def _emb_t(x_t, table_flat):
    T, B = x_t.shape
    info = plsc.get_sparse_core_info()
    nc, ns = info.num_cores, info.num_subcores
    nw = nc * ns
    n_units = T * (B // CB)
    units_per_worker = n_units // nw
    assert units_per_worker * nw == n_units and units_per_worker % 2 == 0
    nchunks = B // CB

    mesh = plsc.VectorSubcoreMesh(core_axis_name="c", subcore_axis_name="s")

    @functools.partial(
        pl.kernel,
        mesh=mesh,
        compiler_params=pltpu.CompilerParams(needs_layout_passes=False),
        out_type=jax.ShapeDtypeStruct((T, D_MODEL, B), jnp.float32),
        scratch_types=[
            pltpu.VMEM((D_MODEL * VOCAB,), jnp.float32),
            pltpu.VMEM((CB,), jnp.int32),
            pltpu.VMEM((CB,), jnp.int32),
            pltpu.VMEM((D_MODEL, CB), jnp.float32),
            pltpu.VMEM((D_MODEL, CB), jnp.float32),
            pltpu.SemaphoreType.DMA,
            pltpu.SemaphoreType.DMA,
            pltpu.SemaphoreType.DMA,
            pltpu.SemaphoreType.DMA,
        ],
    )
    def k(xt_hbm, tab_hbm, out_hbm, tab_v, idx0, idx1, buf0, buf1,
          w0, w1, i0, i1):
        idx = (idx0, idx1)
        buf = (buf0, buf1)
        wsem = (w0, w1)
        isem = (i0, i1)
        wid = lax.axis_index("s") * nc + lax.axis_index("c")
        u_base = wid * units_per_worker
        u_last = u_base + units_per_worker - 1

        pltpu.sync_copy(tab_hbm, tab_v)

        def fire_idx(p, u):
            uc = jnp.minimum(u, u_last)
            t = uc // nchunks
            b0 = (uc % nchunks) * CB
            pltpu.async_copy(xt_hbm.at[t, pl.ds(b0, CB)], idx[p], isem[p])

        def wait_idx(p):
            pltpu.make_async_copy(xt_hbm.at[0, pl.ds(0, CB)], idx[p],
                                  isem[p]).wait()

        def compute(p):
            @plsc.parallel_loop(0, LG)
            def _(g):
                iv = idx[p][pl.ds(g * 16, 16)]
                for d in range(D_MODEL):
                    v = plsc.load_gather(tab_v, [iv + d * VOCAB])
                    buf[p][d, pl.ds(g * 16, 16)] = v

        def fire_write(p, u):
            t = u // nchunks
            b0 = (u % nchunks) * CB
            pltpu.async_copy(buf[p], out_hbm.at[t, :, pl.ds(b0, CB)],
                             wsem[p])

        def wait_write(p):
            pltpu.make_async_copy(buf[p], out_hbm.at[0, :, pl.ds(0, CB)],
                                  wsem[p]).wait()

        fire_idx(0, u_base)
        fire_idx(1, u_base + 1)
        for p in range(2):
            wait_idx(p)
            compute(p)
            fire_write(p, u_base + p)
            fire_idx(p, u_base + p + 2)

        def body(i, _):
            for p in range(2):
                u = u_base + 2 * i + p
                wait_idx(p)
                wait_write(p)
                compute(p)
                fire_write(p, u)
                fire_idx(p, u + 2)
            return ()

        lax.fori_loop(1, units_per_worker // 2, body, ())
        wait_idx(0)
        wait_idx(1)
        wait_write(0)
        wait_write(1)

    return k(x_t, table_flat)


def kernel(x, emb_weight):
    b, t = x.shape
    x_t = x.T
    table_flat = emb_weight.T.reshape(D_MODEL * VOCAB)
    out_t = _emb_t(x_t, table_flat)
    return jnp.transpose(out_t, (2, 0, 1))


# speedup vs baseline: 9.1166x; 1.0002x over previous
import functools

import jax
import jax.numpy as jnp
from jax import lax
from jax.experimental import pallas as pl
from jax.experimental.pallas import tpu as pltpu
from jax.experimental.pallas import tpu_sc as plsc

D_MODEL = 64
VOCAB = 288
CB = 512
LG = CB // 16


def _emb_t(x_t, table_flat):
    T, B = x_t.shape
    info = plsc.get_sparse_core_info()
    nc, ns = info.num_cores, info.num_subcores
    nw = nc * ns
    n_units = T * (B // CB)
    units_per_worker = n_units // nw
    assert units_per_worker * nw == n_units and units_per_worker % 2 == 0
    nchunks = B // CB

    mesh = plsc.VectorSubcoreMesh(core_axis_name="c", subcore_axis_name="s")

    @functools.partial(
        pl.kernel,
        mesh=mesh,
        compiler_params=pltpu.CompilerParams(needs_layout_passes=False),
        out_type=jax.ShapeDtypeStruct((T, D_MODEL, B), jnp.float32),
        scratch_types=[
            pltpu.VMEM((D_MODEL * VOCAB,), jnp.float32),
            pltpu.VMEM((CB,), jnp.int32),
            pltpu.VMEM((CB,), jnp.int32),
            pltpu.VMEM((D_MODEL, CB), jnp.float32),
            pltpu.VMEM((D_MODEL, CB), jnp.float32),
            pltpu.SemaphoreType.DMA,
            pltpu.SemaphoreType.DMA,
            pltpu.SemaphoreType.DMA,
            pltpu.SemaphoreType.DMA,
        ],
    )
    def k(xt_hbm, tab_hbm, out_hbm, tab_v, idx0, idx1, buf0, buf1,
          w0, w1, i0, i1):
        idx = (idx0, idx1)
        buf = (buf0, buf1)
        wsem = (w0, w1)
        isem = (i0, i1)
        wid = lax.axis_index("s") * nc + lax.axis_index("c")
        u_base = wid * units_per_worker
        u_last = u_base + units_per_worker - 1

        pltpu.sync_copy(tab_hbm, tab_v)

        def fire_idx(p, u):
            uc = jnp.minimum(u, u_last)
            t = uc // nchunks
            b0 = (uc % nchunks) * CB
            pltpu.async_copy(xt_hbm.at[t, pl.ds(b0, CB)], idx[p], isem[p])

        def wait_idx(p):
            pltpu.make_async_copy(xt_hbm.at[0, pl.ds(0, CB)], idx[p],
                                  isem[p]).wait()

        def compute(p):
            @plsc.parallel_loop(0, LG)
            def _(g):
                iv = idx[p][pl.ds(g * 16, 16)]
                for d in range(D_MODEL):
                    v = plsc.load_gather(tab_v, [iv + d * VOCAB])
                    buf[p][d, pl.ds(g * 16, 16)] = v

        def fire_write(p, u):
            t = u // nchunks
            b0 = (u % nchunks) * CB
            pltpu.async_copy(buf[p], out_hbm.at[t, :, pl.ds(b0, CB)],
                             wsem[p])

        def wait_write(p):
            pltpu.make_async_copy(buf[p], out_hbm.at[0, :, pl.ds(0, CB)],
                                  wsem[p]).wait()

        fire_idx(0, u_base)
        fire_idx(1, u_base + 1)
        for p in range(2):
            wait_idx(p)
            compute(p)
            fire_write(p, u_base + p)
            fire_idx(p, u_base + p + 2)

        def body(i, _):
            for p in range(2):
                u = u_base + 2 * i + p
                wait_idx(p)
                wait_write(p)
                compute(p)
                fire_write(p, u)
                fire_idx(p, u + 2)
            return ()

        lax.fori_loop(1, units_per_worker // 2, body, ())
        wait_idx(0)
        wait_idx(1)
        wait_write(0)
        wait_write(1)

    return k(x_t, table_flat)


def kernel(x, emb_weight):
    b, t = x.shape
    x_t = x.astype(jnp.int32).T
    table_flat = emb_weight.T.reshape(D_MODEL * VOCAB)
    out_t = _emb_t(x_t, table_flat)
    return jnp.transpose(out_t, (2, 0, 1))
